# baseline (speedup 1.0000x reference)
import functools

import jax
import jax.numpy as jnp
from jax import lax
from jax.experimental import pallas as pl
from jax.experimental.pallas import tpu as pltpu
from jax.experimental.pallas import tpu_sc as plsc

_B, _F = 4096, 26
_DW, _DD = 8, 16
_DIN = _F * _DD
_DPAD = 512
_V = 1000000
_NC, _NS = 2, 16
_NW = _NC * _NS
_BPW = _B // _NW
_IPW = _BPW * _F
_G = 128
_NG = _IPW // _G
_BH = _BPW // 2
_NGH = _NG // 2


def _sc_body(xf_hbm, deep_hbm, wide_hbm, deep_out, wide_out,
             idx_v, dvi_v, wvi_v, dg, wg, buf, wacc, sem_d, sem_w):
    sid = lax.axis_index("s")
    wid = sid * _NC + lax.axis_index("c")
    b0 = wid * _BPW
    pltpu.sync_copy(xf_hbm.at[pl.ds(b0 * _F, _IPW)], idx_v)

    iota = lax.iota(jnp.int32, 16)
    zf = jnp.zeros((16,), jnp.float32)

    def prep(j, _):
        sl = pl.ds(j * 16, 16)
        iv = idx_v[sl]
        dvi_v[sl] = lax.shift_right_logical(iv, 3)
        wvi_v[sl] = lax.shift_right_logical(iv, 4)
        return 0
    lax.fori_loop(0, _IPW // 16, prep, 0)

    def zero_wacc(b, _):
        wacc[b, :] = zf
        return 0
    lax.fori_loop(0, _BPW, zero_wacc, 0)

    def zero_pad(b, _):
        for j in range(_DIN // 16, _DPAD // 16):
            buf[b, pl.ds(j * 16, 16)] = zf
        return 0
    lax.fori_loop(0, _BH, zero_pad, 0)

    def extract(c, slot, h):
        def body(i, _):
            r = c * _G + i
            isp = plsc.load_gather(idx_v, [jnp.full((16,), r, jnp.int32)])
            b = r // _F
            f = r - b * _F
            dcols = ((isp & 7) << 4) + iota
            v = plsc.load_gather(dg.at[slot], [jnp.full((16,), i, jnp.int32), dcols])
            buf[b - h * _BH, pl.ds(f * _DD, _DD)] = v
            wcols = ((isp & 15) << 3) + (iota & 7)
            w = plsc.load_gather(wg.at[slot], [jnp.full((16,), i, jnp.int32), wcols])
            plsc.addupdate(wacc.at[b], w)
            return 0
        lax.fori_loop(0, _G, body, 0)

    def fire(c, slot):
        sl = pl.ds(c * _G, _G)
        dsl = pl.ds(c * _G, _G)
        dcp = pltpu.async_copy(deep_hbm.at[dvi_v.at[dsl]], dg.at[slot], sem_d)
        wcp = pltpu.async_copy(wide_hbm.at[wvi_v.at[dsl]], wg.at[slot], sem_w)
        return dcp, wcp

    pend = fire(0, 0)
    for h in range(2):
        for cc in range(_NGH):
            c = h * _NGH + cc
            slot = c % 2
            if c + 1 < _NG:
                nxt = fire(c + 1, (c + 1) % 2)
            dcp, wcp = pend
            dcp.wait()
            wcp.wait()
            extract(c, slot, h)
            if c + 1 < _NG:
                pend = nxt
        pltpu.sync_copy(buf, deep_out.at[pl.ds(b0 + h * _BH, _BH)])
    pltpu.sync_copy(wacc, wide_out.at[pl.ds(b0, _BPW)])


_sc_gather = functools.partial(
    pl.kernel,
    out_type=(
        jax.ShapeDtypeStruct((_B, _DPAD), jnp.float32),
        jax.ShapeDtypeStruct((_B, 16), jnp.float32),
    ),
    mesh=plsc.VectorSubcoreMesh(
        core_axis_name="c", subcore_axis_name="s", num_cores=_NC, num_subcores=_NS
    ),
    compiler_params=pltpu.CompilerParams(
        use_tc_tiling_on_sc=False, needs_layout_passes=False
    ),
    scratch_types=[
        pltpu.VMEM((_IPW,), jnp.int32),
        pltpu.VMEM((_IPW,), jnp.int32),
        pltpu.VMEM((_IPW,), jnp.int32),
        pltpu.VMEM((2, _G, 128), jnp.float32),
        pltpu.VMEM((2, _G, 128), jnp.float32),
        pltpu.VMEM((_BH, _DPAD), jnp.float32),
        pltpu.VMEM((_BPW, 16), jnp.float32),
        pltpu.SemaphoreType.DMA,
        pltpu.SemaphoreType.DMA,
    ],
)(_sc_body)


_BLK = 512


def _tc_body(deep_ref, wide_ref, g_ref, bta_ref,
             w1, b1, w2, b2, w3, b3, w4, b4, out_ref):
    wide_sum = 0.5 * jnp.sum(wide_ref[...], axis=1, keepdims=True)
    h0 = deep_ref[...]
    mu = jnp.sum(h0, axis=1, keepdims=True) * (1.0 / _DIN)
    var = jnp.sum(h0 * h0, axis=1, keepdims=True) * (1.0 / _DIN) - mu * mu
    h = (h0 - mu) * lax.rsqrt(var + 1e-5) * g_ref[...] + bta_ref[...]
    h = jnp.maximum(jnp.dot(h, w1[...], preferred_element_type=jnp.float32) + b1[...], 0.0)
    h = jnp.maximum(jnp.dot(h, w2[...], preferred_element_type=jnp.float32) + b2[...], 0.0)
    h = jnp.maximum(jnp.dot(h, w3[...], preferred_element_type=jnp.float32) + b3[...], 0.0)
    z = jnp.dot(h, w4[...], preferred_element_type=jnp.float32) + b4[...] + wide_sum
    out_ref[...] = jax.nn.sigmoid(z)


def _full(shape):
    return pl.BlockSpec(shape, lambda i: (0, 0))


_tc_mlp = pl.pallas_call(
    _tc_body,
    grid=(_B // _BLK,),
    in_specs=[
        pl.BlockSpec((_BLK, _DPAD), lambda i: (i, 0)),
        pl.BlockSpec((_BLK, 16), lambda i: (i, 0)),
        _full((1, _DPAD)),
        _full((1, _DPAD)),
        _full((_DPAD, 512)),
        _full((1, 512)),
        _full((512, 256)),
        _full((1, 256)),
        _full((256, 128)),
        _full((1, 128)),
        _full((128, 1)),
        _full((1, 1)),
    ],
    out_specs=pl.BlockSpec((_BLK, 1), lambda i: (i, 0)),
    out_shape=jax.ShapeDtypeStruct((_B, 1), jnp.float32),
)


def kernel(x, wide_table, deep_table, ln_gamma, ln_beta,
           W1, b1, W2, b2, W3, b3, W4, b4):
    xf = x.reshape(_B * _F)
    deep128 = deep_table.reshape(_V * _DD // 128, 128)
    wide128 = wide_table.reshape(_V * _DW // 128, 128)
    deep_cat, wide16 = _sc_gather(xf, deep128, wide128)
    pad = _DPAD - _DIN
    gp = jnp.pad(ln_gamma, (0, pad)).reshape(1, _DPAD)
    bp = jnp.pad(ln_beta, (0, pad)).reshape(1, _DPAD)
    w1p = jnp.pad(W1, ((0, pad), (0, 0)))
    return _tc_mlp(
        deep_cat, wide16, gp, bp,
        w1p, b1.reshape(1, 512), W2, b2.reshape(1, 256),
        W3, b3.reshape(1, 128), W4, b4.reshape(1, 1),
    )


# speedup vs baseline: 1.6454x; 1.6454x over previous
import functools

import jax
import jax.numpy as jnp
from jax import lax
from jax.experimental import pallas as pl
from jax.experimental.pallas import tpu as pltpu
from jax.experimental.pallas import tpu_sc as plsc

_B, _F = 4096, 26
_DW, _DD = 8, 16
_DIN = _F * _DD
_DPAD = 512
_V = 1000000
_NC, _NS = 2, 16
_NW = _NC * _NS
_BPW = _B // _NW
_IPW = _BPW * _F
_RNG = 31232
_TAIL0 = 999936
_WIN = 2048
_NWIN = 16
_CAP = 6656
_GPW = _CAP // 128 + _NWIN + 1
_DRW = _GPW * 16
_WRW = _GPW * 8
_IRW = 72
_MAXW = 16

_mesh = plsc.VectorSubcoreMesh(
    core_axis_name="c", subcore_axis_name="s", num_cores=_NC, num_subcores=_NS
)


def _k1_body(xt_hbm, deep3_hbm, wide2_hbm, tld_hbm, tlw_hbm,
             epd_out, epw_out, ip_out, cnt_out,
             idxc, viv, hii, wviv, whii, deepw, widew, tld_v, tlw_v,
             ebd, ebw, ibw, sv, sem):
    sid = lax.axis_index("s")
    wid = sid * _NC + lax.axis_index("c")
    rw0 = wid * _RNG
    rng = jnp.where(wid == _NW - 1, _V - rw0, _RNG)
    iota = lax.iota(jnp.int32, 16)
    pltpu.sync_copy(tld_hbm, tld_v)
    pltpu.sync_copy(tlw_hbm, tlw_v)

    def scan_wave(start):
        def chunk_body(ch, o):
            pltpu.sync_copy(xt_hbm.at[:, pl.ds(ch * 512, 512)], idxc)
            def qbody(q, o):
                colv26 = (ch * 512 + q * 16 + iota) * _F
                for f in range(_F):
                    iv = idxc[f, pl.ds(q * 16, 16)]
                    m = (iv >= rw0) & (iv < rw0 + rng)
                    pref = plsc.cumsum(jnp.where(m, 1, 0)) - 1
                    pos = o + pref - start
                    mw = m & (pos >= 0) & (pos < _CAP)
                    plsc.store_scatter(viv, [pos], iv, mask=mw)
                    plsc.store_scatter(hii, [pos], colv26 + f, mask=mw)
                    o = o + jnp.max(plsc.all_reduce_population_count(m))
                return o
            return lax.fori_loop(0, 32, qbody, o)
        total = lax.fori_loop(0, 8, chunk_body, jnp.int32(0))
        return total, jnp.clip(total - start, 0, _CAP)

    def window_select(s, wcnt, tail):
        nv = (wcnt + 15) // 16
        def vb(t, o):
            v = viv[pl.ds(t * 16, 16)]
            i = hii[pl.ds(t * 16, 16)]
            if tail:
                m = v >= _TAIL0
            else:
                m = (((v - rw0) >> 11) == s) & (v < _TAIL0)
            m = m & (t * 16 + iota < wcnt)
            pref = plsc.cumsum(jnp.where(m, 1, 0)) - 1
            pos = o + pref
            plsc.store_scatter(wviv, [pos], v, mask=m)
            plsc.store_scatter(whii, [pos], i, mask=m)
            return o + jnp.max(plsc.all_reduce_population_count(m))
        return lax.fori_loop(0, nv, vb, jnp.int32(0))

    def extract_groups(off, wcnt, w, gw, tail):
        ng = (wcnt + 127) // 128
        def grp(g, gw):
            base = g * 128
            for j in range(16):
                for k in range(8):
                    h = jnp.minimum(base + j * 8 + k, wcnt - 1)
                    hsp = plsc.load_gather(wviv, [jnp.full((16,), 0, jnp.int32) + h])
                    c = hsp - off
                    if tail:
                        ve = plsc.load_gather(tld_v, [iota >> 3, iota & 7, c])
                    else:
                        ve = plsc.load_gather(deepw, [iota >> 3, iota & 7, c])
                    ebd[j, pl.ds(k * 16, 16)] = ve
            for j in range(8):
                for k in range(8):
                    h2 = jnp.minimum(base + j * 16 + k * 2 + (iota >> 3), wcnt - 1)
                    hv = plsc.load_gather(wviv, [h2])
                    c2 = hv - off
                    if tail:
                        wv = plsc.load_gather(tlw_v, [iota & 7, c2])
                    else:
                        wv = plsc.load_gather(widew, [iota & 7, c2])
                    ebw[j, pl.ds(k * 16, 16)] = wv
            for q in range(8):
                hq = jnp.minimum(base + q * 16 + iota, wcnt - 1)
                ivals = plsc.load_gather(whii, [hq])
                plsc.store_scatter(
                    ibw, [jnp.full((16,), 0, jnp.int32) + gw, q * 16 + iota],
                    ivals)
            pltpu.sync_copy(ebd, epd_out.at[wid, pl.ds(w * _DRW + gw * 16, 16)])
            pltpu.sync_copy(ebw, epw_out.at[wid, pl.ds(w * _WRW + gw * 8, 8)])
            return gw + 1
        return lax.fori_loop(0, ng, grp, gw)

    def wave_body(carry):
        w, total, cntv = carry
        total, wcnt = scan_wave(w * _CAP)
        def win_body(s, gw):
            off = jnp.minimum(rw0 + s * _WIN, _TAIL0 - _WIN)
            c = window_select(s, wcnt, False)
            pltpu.sync_copy(deep3_hbm.at[:, :, pl.ds(off, _WIN)], deepw)
            pltpu.sync_copy(wide2_hbm.at[:, pl.ds(off, _WIN)], widew)
            return extract_groups(off, c, w, gw, False)
        gw = lax.fori_loop(0, _NWIN, win_body, jnp.int32(0))
        ct = window_select(0, wcnt, True)
        gw = extract_groups(_TAIL0, ct, w, gw, True)
        pltpu.sync_copy(ibw, ip_out.at[wid, pl.ds(w * _IRW, _IRW)])
        cntv = jnp.where(iota == w, gw, cntv)
        return w + 1, total, cntv

    def wave_cond(carry):
        w, total, _ = carry
        return (w == 0) | (w * _CAP < total)

    _, _, cntv = lax.while_loop(
        wave_cond, wave_body,
        (jnp.int32(0), jnp.int32(0), jnp.zeros((16,), jnp.int32)))
    sv[...] = cntv
    pltpu.sync_copy(sv, cnt_out.at[wid])


_k1 = functools.partial(
    pl.kernel,
    out_type=(
        jax.ShapeDtypeStruct((_NW, _MAXW * _DRW, 128), jnp.float32),
        jax.ShapeDtypeStruct((_NW, _MAXW * _WRW, 128), jnp.float32),
        jax.ShapeDtypeStruct((_NW, _MAXW * _IRW, 128), jnp.int32),
        jax.ShapeDtypeStruct((_NW, 16), jnp.int32),
    ),
    mesh=_mesh,
    compiler_params=pltpu.CompilerParams(
        use_tc_tiling_on_sc=True, needs_layout_passes=False
    ),
    scratch_types=[
        pltpu.VMEM((_F, 512), jnp.int32),
        pltpu.VMEM((_CAP,), jnp.int32),
        pltpu.VMEM((_CAP,), jnp.int32),
        pltpu.VMEM((_CAP,), jnp.int32),
        pltpu.VMEM((_CAP,), jnp.int32),
        pltpu.VMEM((2, 8, _WIN), jnp.float32),
        pltpu.VMEM((8, _WIN), jnp.float32),
        pltpu.VMEM((2, 8, 64), jnp.float32),
        pltpu.VMEM((8, 64), jnp.float32),
        pltpu.VMEM((16, 128), jnp.float32),
        pltpu.VMEM((8, 128), jnp.float32),
        pltpu.VMEM((_IRW, 128), jnp.int32),
        pltpu.VMEM((16,), jnp.int32),
        pltpu.SemaphoreType.DMA,
    ],
)(_k1_body)


def _k2_body(epd_hbm, epw_hbm, ip_hbm, cnt_hbm, out16, wout16,
             cv, ebd, ebw, ridx, rows_v, wrows_v, sem):
    sid = lax.axis_index("s")
    wid = sid * _NC + lax.axis_index("c")
    pltpu.sync_copy(cnt_hbm.at[wid], cv)
    iota = lax.iota(jnp.int32, 16)

    def wave(w, _):
        ng = jnp.max(plsc.load_gather(cv, [jnp.full((16,), 0, jnp.int32) + w]))
        def grp(g, _):
            pltpu.sync_copy(epd_hbm.at[wid, pl.ds(w * _DRW + g * 16, 16)], ebd)
            pltpu.sync_copy(epw_hbm.at[wid, pl.ds(w * _WRW + g * 8, 8)], ebw)
            pltpu.sync_copy(ip_hbm.at[wid, w * _IRW + g], ridx)
            for jr in range(16):
                for k in range(8):
                    rows_v[jr * 8 + k, :] = ebd[jr, pl.ds(k * 16, 16)]
            for jr in range(8):
                for k in range(16):
                    wrows_v[jr * 16 + k, :] = plsc.load_gather(
                        ebw, [jnp.full((16,), jr, jnp.int32), k * 8 + (iota & 7)])
            pltpu.async_copy(rows_v, out16.at[ridx], sem).wait()
            pltpu.async_copy(wrows_v, wout16.at[ridx], sem).wait()
            return 0
        lax.fori_loop(0, ng, grp, 0)
        return 0
    lax.fori_loop(0, _MAXW, wave, 0)


_k2 = functools.partial(
    pl.kernel,
    out_type=(
        jax.ShapeDtypeStruct((_B * _F, 16), jnp.float32),
        jax.ShapeDtypeStruct((_B * _F, 16), jnp.float32),
    ),
    mesh=_mesh,
    compiler_params=pltpu.CompilerParams(
        use_tc_tiling_on_sc=False, needs_layout_passes=False
    ),
    scratch_types=[
        pltpu.VMEM((16,), jnp.int32),
        pltpu.VMEM((16, 128), jnp.float32),
        pltpu.VMEM((8, 128), jnp.float32),
        pltpu.VMEM((128,), jnp.int32),
        pltpu.VMEM((128, 16), jnp.float32),
        pltpu.VMEM((128, 16), jnp.float32),
        pltpu.SemaphoreType.DMA,
    ],
)(_k2_body)


def _k3_body(d16_hbm, w16_hbm, deep_out, wide_out,
             dhalf, whalf, buf, wacc, sem):
    sid = lax.axis_index("s")
    wid = sid * _NC + lax.axis_index("c")
    zf = jnp.zeros((16,), jnp.float32)

    def zero_pad(b, _):
        for j in range(_DIN // 16, _DPAD // 16):
            buf[b, pl.ds(j * 16, 16)] = zf
        return 0
    lax.fori_loop(0, 64, zero_pad, 0)

    for h in range(2):
        pltpu.sync_copy(d16_hbm.at[pl.ds(wid * _IPW + h * 1664, 1664)], dhalf)
        pltpu.sync_copy(w16_hbm.at[pl.ds(wid * _IPW + h * 1664, 1664)], whalf)
        def relayout(b, _):
            acc = whalf[b * _F, :]
            for f in range(_F):
                buf[b, pl.ds(f * _DD, _DD)] = dhalf[b * _F + f, :]
                if f:
                    acc = acc + whalf[b * _F + f, :]
            wacc[h * 64 + b, :] = acc
            return 0
        lax.fori_loop(0, 64, relayout, 0)
        pltpu.sync_copy(buf, deep_out.at[pl.ds(wid * _BPW + h * 64, 64)])
    pltpu.sync_copy(wacc, wide_out.at[pl.ds(wid * _BPW, _BPW)])


_k3 = functools.partial(
    pl.kernel,
    out_type=(
        jax.ShapeDtypeStruct((_B, _DPAD), jnp.float32),
        jax.ShapeDtypeStruct((_B, 16), jnp.float32),
    ),
    mesh=_mesh,
    compiler_params=pltpu.CompilerParams(
        use_tc_tiling_on_sc=False, needs_layout_passes=False
    ),
    scratch_types=[
        pltpu.VMEM((1664, 16), jnp.float32),
        pltpu.VMEM((1664, 16), jnp.float32),
        pltpu.VMEM((64, _DPAD), jnp.float32),
        pltpu.VMEM((_BPW, 16), jnp.float32),
        pltpu.SemaphoreType.DMA,
    ],
)(_k3_body)


_BLK = 512


def _tc_body(deep_ref, wide_ref, g_ref, bta_ref,
             w1, b1, w2, b2, w3, b3, w4, b4, out_ref):
    wide_sum = 0.5 * jnp.sum(wide_ref[...], axis=1, keepdims=True)
    h0 = deep_ref[...]
    mu = jnp.sum(h0, axis=1, keepdims=True) * (1.0 / _DIN)
    var = jnp.sum(h0 * h0, axis=1, keepdims=True) * (1.0 / _DIN) - mu * mu
    h = (h0 - mu) * lax.rsqrt(var + 1e-5) * g_ref[...] + bta_ref[...]
    h = jnp.maximum(jnp.dot(h, w1[...], preferred_element_type=jnp.float32) + b1[...], 0.0)
    h = jnp.maximum(jnp.dot(h, w2[...], preferred_element_type=jnp.float32) + b2[...], 0.0)
    h = jnp.maximum(jnp.dot(h, w3[...], preferred_element_type=jnp.float32) + b3[...], 0.0)
    z = jnp.dot(h, w4[...], preferred_element_type=jnp.float32) + b4[...] + wide_sum
    out_ref[...] = jax.nn.sigmoid(z)


def _full(shape):
    return pl.BlockSpec(shape, lambda i: (0, 0))


_tc_mlp = pl.pallas_call(
    _tc_body,
    grid=(_B // _BLK,),
    in_specs=[
        pl.BlockSpec((_BLK, _DPAD), lambda i: (i, 0)),
        pl.BlockSpec((_BLK, 16), lambda i: (i, 0)),
        _full((1, _DPAD)),
        _full((1, _DPAD)),
        _full((_DPAD, 512)),
        _full((1, 512)),
        _full((512, 256)),
        _full((1, 256)),
        _full((256, 128)),
        _full((1, 128)),
        _full((128, 1)),
        _full((1, 1)),
    ],
    out_specs=pl.BlockSpec((_BLK, 1), lambda i: (i, 0)),
    out_shape=jax.ShapeDtypeStruct((_B, 1), jnp.float32),
)


def kernel(x, wide_table, deep_table, ln_gamma, ln_beta,
           W1, b1, W2, b2, W3, b3, W4, b4):
    xt = x.T
    deep3 = deep_table.T.reshape(2, 8, _V)
    wide2 = wide_table.T
    tld = deep_table[_V - 64:, :].T.reshape(2, 8, 64)
    tlw = wide_table[_V - 64:, :].T
    epd, epw, ip, cnts = _k1(xt, deep3, wide2, tld, tlw)
    d16, w16 = _k2(epd, epw, ip, cnts)
    deep_cat, wide16 = _k3(d16, w16)
    pad = _DPAD - _DIN
    gp = jnp.pad(ln_gamma, (0, pad)).reshape(1, _DPAD)
    bp = jnp.pad(ln_beta, (0, pad)).reshape(1, _DPAD)
    w1p = jnp.pad(W1, ((0, pad), (0, 0)))
    return _tc_mlp(
        deep_cat, wide16, gp, bp,
        w1p, b1.reshape(1, 512), W2, b2.reshape(1, 256),
        W3, b3.reshape(1, 128), W4, b4.reshape(1, 1),
    )


# speedup vs baseline: 1.9349x; 1.1759x over previous
import functools

import jax
import jax.numpy as jnp
from jax import lax
from jax.experimental import pallas as pl
from jax.experimental.pallas import tpu as pltpu
from jax.experimental.pallas import tpu_sc as plsc

_B, _F = 4096, 26
_DW, _DD = 8, 16
_DIN = _F * _DD
_DPAD = 512
_V = 1000000
_NC, _NS = 2, 16
_NW = _NC * _NS
_BPW = _B // _NW
_IPW = _BPW * _F
_RNG = 31232
_TAIL0 = 999936
_WIN = 2048
_NWIN = 16
_CAP = 6656
_GPW = _CAP // 128 + _NWIN + 1
_DRW = _GPW * 16
_WRW = _GPW * 8
_IRW = 72
_MAXW = 16

_mesh = plsc.VectorSubcoreMesh(
    core_axis_name="c", subcore_axis_name="s", num_cores=_NC, num_subcores=_NS
)


def _k1_body(xt_hbm, deep3_hbm, wide2_hbm, tld_hbm, tlw_hbm,
             epd_out, epw_out, ip_out, cnt_out,
             idxc, viv, hii, wviv, whii, deepw, widew, tld_v, tlw_v,
             ebd, ebw, ibw, sv, sem):
    sid = lax.axis_index("s")
    wid = sid * _NC + lax.axis_index("c")
    rw0 = wid * _RNG
    rng = jnp.where(wid == _NW - 1, _V - rw0, _RNG)
    iota = lax.iota(jnp.int32, 16)
    pltpu.sync_copy(tld_hbm, tld_v)
    pltpu.sync_copy(tlw_hbm, tlw_v)

    def scan_wave(start):
        o = jnp.int32(0)
        cps = {}
        cps[0] = pltpu.async_copy(
            xt_hbm.at[:, pl.ds(0, 512)], idxc.at[0], sem)
        for ch in range(8):
            sl = ch % 2
            if ch + 1 < 8:
                cps[(ch + 1) % 2] = pltpu.async_copy(
                    xt_hbm.at[:, pl.ds((ch + 1) * 512, 512)],
                    idxc.at[(ch + 1) % 2], sem)
            cps[sl].wait()
            def qbody(q, o, ch=ch, sl=sl):
                colv26 = (ch * 512 + q * 16 + iota) * _F
                for f in range(_F):
                    iv = idxc[sl, f, pl.ds(q * 16, 16)]
                    m = (iv >= rw0) & (iv < rw0 + rng)
                    pref = plsc.cumsum(jnp.where(m, 1, 0)) - 1
                    pos = o + pref - start
                    mw = m & (pos >= 0) & (pos < _CAP)
                    plsc.store_scatter(viv, [pos], iv, mask=mw)
                    plsc.store_scatter(hii, [pos], colv26 + f, mask=mw)
                    o = o + jnp.max(plsc.all_reduce_population_count(m))
                return o
            o = lax.fori_loop(0, 32, qbody, o)
        total = o
        return total, jnp.clip(total - start, 0, _CAP)

    def window_select(s, wcnt, tail):
        nv = (wcnt + 15) // 16
        def vb(t, o):
            v = viv[pl.ds(t * 16, 16)]
            i = hii[pl.ds(t * 16, 16)]
            if tail:
                m = v >= _TAIL0
            else:
                m = (((v - rw0) >> 11) == s) & (v < _TAIL0)
            m = m & (t * 16 + iota < wcnt)
            pref = plsc.cumsum(jnp.where(m, 1, 0)) - 1
            pos = o + pref
            plsc.store_scatter(wviv, [pos], v, mask=m)
            plsc.store_scatter(whii, [pos], i, mask=m)
            return o + jnp.max(plsc.all_reduce_population_count(m))
        return lax.fori_loop(0, nv, vb, jnp.int32(0))

    def extract_groups(off, wcnt, w, gw, tail):
        ng = (wcnt + 127) // 128
        def grp(g, gw):
            base = g * 128
            for j in range(16):
                for k in range(8):
                    h = jnp.minimum(base + j * 8 + k, wcnt - 1)
                    hsp = plsc.load_gather(wviv, [jnp.full((16,), 0, jnp.int32) + h])
                    c = hsp - off
                    if tail:
                        ve = plsc.load_gather(tld_v, [iota >> 3, iota & 7, c])
                    else:
                        ve = plsc.load_gather(deepw, [iota >> 3, iota & 7, c])
                    ebd[j, pl.ds(k * 16, 16)] = ve
            for j in range(8):
                for k in range(8):
                    h2 = jnp.minimum(base + j * 16 + k * 2 + (iota >> 3), wcnt - 1)
                    hv = plsc.load_gather(wviv, [h2])
                    c2 = hv - off
                    if tail:
                        wv = plsc.load_gather(tlw_v, [iota & 7, c2])
                    else:
                        wv = plsc.load_gather(widew, [iota & 7, c2])
                    ebw[j, pl.ds(k * 16, 16)] = wv
            for q in range(8):
                hq = jnp.minimum(base + q * 16 + iota, wcnt - 1)
                ivals = plsc.load_gather(whii, [hq])
                plsc.store_scatter(
                    ibw, [jnp.full((16,), 0, jnp.int32) + gw, q * 16 + iota],
                    ivals)
            pltpu.sync_copy(ebd, epd_out.at[wid, pl.ds(w * _DRW + gw * 16, 16)])
            pltpu.sync_copy(ebw, epw_out.at[wid, pl.ds(w * _WRW + gw * 8, 8)])
            return gw + 1
        return lax.fori_loop(0, ng, grp, gw)

    def wave_body(carry):
        w, total, cntv = carry
        total, wcnt = scan_wave(w * _CAP)
        def win_body(s, gw):
            off = jnp.minimum(rw0 + s * _WIN, _TAIL0 - _WIN)
            dcp = pltpu.async_copy(
                deep3_hbm.at[:, :, pl.ds(off, _WIN)], deepw, sem)
            wcp = pltpu.async_copy(
                wide2_hbm.at[:, pl.ds(off, _WIN)], widew, sem)
            c = window_select(s, wcnt, False)
            dcp.wait()
            wcp.wait()
            return extract_groups(off, c, w, gw, False)
        gw = lax.fori_loop(0, _NWIN, win_body, jnp.int32(0))
        ct = window_select(0, wcnt, True)
        gw = extract_groups(_TAIL0, ct, w, gw, True)
        pltpu.sync_copy(ibw, ip_out.at[wid, pl.ds(w * _IRW, _IRW)])
        cntv = jnp.where(iota == w, gw, cntv)
        return w + 1, total, cntv

    def wave_cond(carry):
        w, total, _ = carry
        return (w == 0) | (w * _CAP < total)

    _, _, cntv = lax.while_loop(
        wave_cond, wave_body,
        (jnp.int32(0), jnp.int32(0), jnp.zeros((16,), jnp.int32)))
    sv[...] = cntv
    pltpu.sync_copy(sv, cnt_out.at[wid])


_k1 = functools.partial(
    pl.kernel,
    out_type=(
        jax.ShapeDtypeStruct((_NW, _MAXW * _DRW, 128), jnp.float32),
        jax.ShapeDtypeStruct((_NW, _MAXW * _WRW, 128), jnp.float32),
        jax.ShapeDtypeStruct((_NW, _MAXW * _IRW, 128), jnp.int32),
        jax.ShapeDtypeStruct((_NW, 16), jnp.int32),
    ),
    mesh=_mesh,
    compiler_params=pltpu.CompilerParams(
        use_tc_tiling_on_sc=True, needs_layout_passes=False
    ),
    scratch_types=[
        pltpu.VMEM((2, _F, 512), jnp.int32),
        pltpu.VMEM((_CAP,), jnp.int32),
        pltpu.VMEM((_CAP,), jnp.int32),
        pltpu.VMEM((_CAP,), jnp.int32),
        pltpu.VMEM((_CAP,), jnp.int32),
        pltpu.VMEM((2, 8, _WIN), jnp.float32),
        pltpu.VMEM((8, _WIN), jnp.float32),
        pltpu.VMEM((2, 8, 64), jnp.float32),
        pltpu.VMEM((8, 64), jnp.float32),
        pltpu.VMEM((16, 128), jnp.float32),
        pltpu.VMEM((8, 128), jnp.float32),
        pltpu.VMEM((_IRW, 128), jnp.int32),
        pltpu.VMEM((16,), jnp.int32),
        pltpu.SemaphoreType.DMA,
    ],
)(_k1_body)


def _k2_body(epd_hbm, epw_hbm, ip_hbm, cnt_hbm, out16, wout16,
             cv, ebd, ebw, ridx, rows_v, wrows_v, sem):
    sid = lax.axis_index("s")
    wid = sid * _NC + lax.axis_index("c")
    pltpu.sync_copy(cnt_hbm.at[wid], cv)
    iota = lax.iota(jnp.int32, 16)

    def wave(w, _):
        ng = jnp.max(plsc.load_gather(cv, [jnp.full((16,), 0, jnp.int32) + w]))
        def grp(g, _):
            c1 = pltpu.async_copy(
                epd_hbm.at[wid, pl.ds(w * _DRW + g * 16, 16)], ebd, sem)
            c2 = pltpu.async_copy(
                epw_hbm.at[wid, pl.ds(w * _WRW + g * 8, 8)], ebw, sem)
            c3 = pltpu.async_copy(ip_hbm.at[wid, w * _IRW + g], ridx, sem)
            c1.wait()
            c2.wait()
            c3.wait()
            for jr in range(16):
                for k in range(8):
                    rows_v[jr * 8 + k, :] = ebd[jr, pl.ds(k * 16, 16)]
            for jr in range(8):
                for k in range(16):
                    wrows_v[jr * 16 + k, :] = plsc.load_gather(
                        ebw, [jnp.full((16,), jr, jnp.int32), k * 8 + (iota & 7)])
            pltpu.async_copy(rows_v, out16.at[ridx], sem).wait()
            pltpu.async_copy(wrows_v, wout16.at[ridx], sem).wait()
            return 0
        lax.fori_loop(0, ng, grp, 0)
        return 0
    lax.fori_loop(0, _MAXW, wave, 0)


_k2 = functools.partial(
    pl.kernel,
    out_type=(
        jax.ShapeDtypeStruct((_B * _F, 16), jnp.float32),
        jax.ShapeDtypeStruct((_B * _F, 16), jnp.float32),
    ),
    mesh=_mesh,
    compiler_params=pltpu.CompilerParams(
        use_tc_tiling_on_sc=False, needs_layout_passes=False
    ),
    scratch_types=[
        pltpu.VMEM((16,), jnp.int32),
        pltpu.VMEM((16, 128), jnp.float32),
        pltpu.VMEM((8, 128), jnp.float32),
        pltpu.VMEM((128,), jnp.int32),
        pltpu.VMEM((128, 16), jnp.float32),
        pltpu.VMEM((128, 16), jnp.float32),
        pltpu.SemaphoreType.DMA,
    ],
)(_k2_body)


def _k3_body(d16_hbm, w16_hbm, deep_out, wide_out,
             dhalf, whalf, buf, wacc, sem):
    sid = lax.axis_index("s")
    wid = sid * _NC + lax.axis_index("c")
    zf = jnp.zeros((16,), jnp.float32)

    def zero_pad(b, _):
        for j in range(_DIN // 16, _DPAD // 16):
            buf[b, pl.ds(j * 16, 16)] = zf
        return 0
    lax.fori_loop(0, 64, zero_pad, 0)

    for h in range(2):
        pltpu.sync_copy(d16_hbm.at[pl.ds(wid * _IPW + h * 1664, 1664)], dhalf)
        pltpu.sync_copy(w16_hbm.at[pl.ds(wid * _IPW + h * 1664, 1664)], whalf)
        def relayout(b, _):
            acc = whalf[b * _F, :]
            for f in range(_F):
                buf[b, pl.ds(f * _DD, _DD)] = dhalf[b * _F + f, :]
                if f:
                    acc = acc + whalf[b * _F + f, :]
            wacc[h * 64 + b, :] = acc
            return 0
        lax.fori_loop(0, 64, relayout, 0)
        pltpu.sync_copy(buf, deep_out.at[pl.ds(wid * _BPW + h * 64, 64)])
    pltpu.sync_copy(wacc, wide_out.at[pl.ds(wid * _BPW, _BPW)])


_k3 = functools.partial(
    pl.kernel,
    out_type=(
        jax.ShapeDtypeStruct((_B, _DPAD), jnp.float32),
        jax.ShapeDtypeStruct((_B, 16), jnp.float32),
    ),
    mesh=_mesh,
    compiler_params=pltpu.CompilerParams(
        use_tc_tiling_on_sc=False, needs_layout_passes=False
    ),
    scratch_types=[
        pltpu.VMEM((1664, 16), jnp.float32),
        pltpu.VMEM((1664, 16), jnp.float32),
        pltpu.VMEM((64, _DPAD), jnp.float32),
        pltpu.VMEM((_BPW, 16), jnp.float32),
        pltpu.SemaphoreType.DMA,
    ],
)(_k3_body)


_BLK = 512


def _tc_body(deep_ref, wide_ref, g_ref, bta_ref,
             w1, b1, w2, b2, w3, b3, w4, b4, out_ref):
    wide_sum = 0.5 * jnp.sum(wide_ref[...], axis=1, keepdims=True)
    h0 = deep_ref[...]
    mu = jnp.sum(h0, axis=1, keepdims=True) * (1.0 / _DIN)
    var = jnp.sum(h0 * h0, axis=1, keepdims=True) * (1.0 / _DIN) - mu * mu
    h = (h0 - mu) * lax.rsqrt(var + 1e-5) * g_ref[...] + bta_ref[...]
    h = jnp.maximum(jnp.dot(h, w1[...], preferred_element_type=jnp.float32) + b1[...], 0.0)
    h = jnp.maximum(jnp.dot(h, w2[...], preferred_element_type=jnp.float32) + b2[...], 0.0)
    h = jnp.maximum(jnp.dot(h, w3[...], preferred_element_type=jnp.float32) + b3[...], 0.0)
    z = jnp.dot(h, w4[...], preferred_element_type=jnp.float32) + b4[...] + wide_sum
    out_ref[...] = jax.nn.sigmoid(z)


def _full(shape):
    return pl.BlockSpec(shape, lambda i: (0, 0))


_tc_mlp = pl.pallas_call(
    _tc_body,
    grid=(_B // _BLK,),
    in_specs=[
        pl.BlockSpec((_BLK, _DPAD), lambda i: (i, 0)),
        pl.BlockSpec((_BLK, 16), lambda i: (i, 0)),
        _full((1, _DPAD)),
        _full((1, _DPAD)),
        _full((_DPAD, 512)),
        _full((1, 512)),
        _full((512, 256)),
        _full((1, 256)),
        _full((256, 128)),
        _full((1, 128)),
        _full((128, 1)),
        _full((1, 1)),
    ],
    out_specs=pl.BlockSpec((_BLK, 1), lambda i: (i, 0)),
    out_shape=jax.ShapeDtypeStruct((_B, 1), jnp.float32),
)


def kernel(x, wide_table, deep_table, ln_gamma, ln_beta,
           W1, b1, W2, b2, W3, b3, W4, b4):
    xt = x.T
    deep3 = deep_table.T.reshape(2, 8, _V)
    wide2 = wide_table.T
    tld = deep_table[_V - 64:, :].T.reshape(2, 8, 64)
    tlw = wide_table[_V - 64:, :].T
    epd, epw, ip, cnts = _k1(xt, deep3, wide2, tld, tlw)
    d16, w16 = _k2(epd, epw, ip, cnts)
    deep_cat, wide16 = _k3(d16, w16)
    pad = _DPAD - _DIN
    gp = jnp.pad(ln_gamma, (0, pad)).reshape(1, _DPAD)
    bp = jnp.pad(ln_beta, (0, pad)).reshape(1, _DPAD)
    w1p = jnp.pad(W1, ((0, pad), (0, 0)))
    return _tc_mlp(
        deep_cat, wide16, gp, bp,
        w1p, b1.reshape(1, 512), W2, b2.reshape(1, 256),
        W3, b3.reshape(1, 128), W4, b4.reshape(1, 1),
    )
